# Initial kernel scaffold; baseline (speedup 1.0000x reference)
import jax
import jax.numpy as jnp
from jax.experimental import pallas as pl


def kernel(x, edge_index, edge_weight, W1a, b1a, g1a, be1a, W2a, b2a, g2a, be2a, W1b, b1b, g1b, be1b, W2b, b2b, g2b, be2b, Wf, bf):
    raise NotImplementedError("write your pallas kernel here")


# speedup vs baseline: 4.0729x; 4.0729x over previous
import functools

import jax
import jax.numpy as jnp
from jax import lax
from jax.experimental import pallas as pl
from jax.experimental.pallas import tpu as pltpu
from jax.experimental.pallas import tpu_sc as plsc

N = 50000
NC = 7
H = 128
E = 800000
BN_EPS = 1e-5

NP = 50176
BN_ROWS = 512
GRID = NP // BN_ROWS

NPA = 51200
TRASH = NP
EBLK = 128
GB = 8
NWORK = 32
BPW = 200
EP = NWORK * BPW * EBLK
ZR = 128
RPT_ZERO = NPA // 16
RPT_OUT = NP // 16
WCH = 112


@functools.cache
def _make_agg(fc, n_tab):
    mesh = plsc.VectorSubcoreMesh(core_axis_name="c", subcore_axis_name="s",
                                  num_cores=2, num_subcores=16)
    out_type = tuple(
        jax.ShapeDtypeStruct((2 * NP, fc), jnp.float32) for _ in range(n_tab)
    )
    scratch = [
        pltpu.VMEM((GB, EBLK), jnp.int32),
        pltpu.VMEM((GB, EBLK), jnp.int32),
        pltpu.VMEM((EBLK, fc), jnp.float32),
        pltpu.VMEM((EBLK, fc), jnp.float32),
        pltpu.VMEM((WCH, fc), jnp.float32),
        pltpu.VMEM((ZR, fc), jnp.float32),
        pltpu.VMEM_SHARED((NPA, fc), jnp.float32),
        pltpu.SemaphoreType.DMA,
        pltpu.SemaphoreType.DMA,
    ]

    @functools.partial(pl.kernel, mesh=mesh, out_type=out_type,
                       scratch_types=scratch,
                       compiler_params=pltpu.CompilerParams(
                           use_tc_tiling_on_sc=False))
    def agg(src2d, dst2d, zrows, *rest):
        tabs = rest[:n_tab]
        outs = rest[n_tab:2 * n_tab]
        sbuf, dbuf, rows0, rows1, obuf, zbuf, acc, sem0, sem1 = rest[2 * n_tab:]
        rowbufs = (rows0, rows1)
        sems = (sem0, sem1)
        c = lax.axis_index("c")
        s = lax.axis_index("s")
        w = s * 2 + c
        pltpu.sync_copy(zrows, zbuf)
        for t in range(n_tab):
            def zbody(k, _):
                pltpu.sync_copy(zbuf, acc.at[pl.ds(s * RPT_ZERO + k * ZR, ZR)])
                return 0
            lax.fori_loop(0, RPT_ZERO // ZR, zbody, 0, unroll=False)
            plsc.subcore_barrier()

            def gbody(g, _):
                base = w * BPW + g * GB
                pltpu.sync_copy(src2d.at[pl.ds(base, GB)], sbuf)
                pltpu.sync_copy(dst2d.at[pl.ds(base, GB)], dbuf)
                descs = [None, None]
                descs[0] = pltpu.async_copy(tabs[t].at[sbuf.at[0]],
                                            rowbufs[0], sems[0])
                for j in range(GB):
                    if j + 1 < GB:
                        descs[(j + 1) % 2] = pltpu.async_copy(
                            tabs[t].at[sbuf.at[j + 1]],
                            rowbufs[(j + 1) % 2], sems[(j + 1) % 2])
                    descs[j % 2].wait()
                    pltpu.sync_copy(rowbufs[j % 2], acc.at[dbuf.at[j]],
                                    add=True)
                return 0
            lax.fori_loop(0, BPW // GB, gbody, 0, unroll=False)
            plsc.subcore_barrier()

            def obody(k, _):
                r0 = s * RPT_OUT + k * WCH
                pltpu.sync_copy(acc.at[pl.ds(r0, WCH)], obuf)
                pltpu.sync_copy(obuf, outs[t].at[pl.ds(c * NP + r0, WCH)])
                return 0
            lax.fori_loop(0, RPT_OUT // WCH, obody, 0, unroll=False)
            plsc.subcore_barrier()

    return agg


def _row_mask(i, h):
    rid = i * BN_ROWS + lax.broadcasted_iota(jnp.int32, (BN_ROWS, 1), 0)
    return jnp.where(rid < N, h, 0.0)


def _stats_update(i, z, s_ref):
    su = jnp.sum(z, axis=0, keepdims=True)
    sq = jnp.sum(z * z, axis=0, keepdims=True)
    upd = jnp.concatenate([su, sq, jnp.zeros((6, H), jnp.float32)], axis=0)

    @pl.when(i == 0)
    def _():
        s_ref[...] = upd

    @pl.when(i != 0)
    def _():
        s_ref[...] = s_ref[...] + upd


def _bn_coefs(s_ref, g_ref, be_ref):
    su = s_ref[0, :]
    sq = s_ref[1, :]
    m = su / N
    v = sq / N - m * m
    scale = g_ref[0, :] * lax.rsqrt(v + BN_EPS)
    shift = be_ref[0, :] - m * scale
    return scale, shift


def _k1_body(x_ref, p0_ref, p1_ref, w_ref, z_ref, s_ref):
    i = pl.program_id(0)
    h = x_ref[...] + p0_ref[...] + p1_ref[...]
    z = lax.dot_general(h, w_ref[...], (((1,), (0,)), ((), ())),
                        preferred_element_type=jnp.float32)
    z_ref[...] = z
    _stats_update(i, z, s_ref)


def _k1(x8, p0, p1, w8):
    return pl.pallas_call(
        _k1_body,
        grid=(GRID,),
        in_specs=[
            pl.BlockSpec((BN_ROWS, 8), lambda i: (i, 0)),
            pl.BlockSpec((BN_ROWS, 8), lambda i: (i, 0)),
            pl.BlockSpec((BN_ROWS, 8), lambda i: (i, 0)),
            pl.BlockSpec((8, H), lambda i: (0, 0)),
        ],
        out_specs=[
            pl.BlockSpec((BN_ROWS, H), lambda i: (i, 0)),
            pl.BlockSpec((8, H), lambda i: (0, 0)),
        ],
        out_shape=[
            jax.ShapeDtypeStruct((NP, H), jnp.float32),
            jax.ShapeDtypeStruct((8, H), jnp.float32),
        ],
    )(x8, p0, p1, w8)


def _bn_mm_body(z_ref, s_ref, g_ref, be_ref, w_ref, z2_ref, s2_ref):
    i = pl.program_id(0)
    scale, shift = _bn_coefs(s_ref, g_ref, be_ref)
    h = jnp.maximum(z_ref[...] * scale + shift, 0.0)
    h = _row_mask(i, h)
    z2 = lax.dot_general(h, w_ref[...], (((1,), (0,)), ((), ())),
                         preferred_element_type=jnp.float32)
    z2_ref[...] = z2
    _stats_update(i, z2, s2_ref)


def _bn_mm(z, s, g, be, w):
    return pl.pallas_call(
        _bn_mm_body,
        grid=(GRID,),
        in_specs=[
            pl.BlockSpec((BN_ROWS, H), lambda i: (i, 0)),
            pl.BlockSpec((8, H), lambda i: (0, 0)),
            pl.BlockSpec((1, H), lambda i: (0, 0)),
            pl.BlockSpec((1, H), lambda i: (0, 0)),
            pl.BlockSpec((H, H), lambda i: (0, 0)),
        ],
        out_specs=[
            pl.BlockSpec((BN_ROWS, H), lambda i: (i, 0)),
            pl.BlockSpec((8, H), lambda i: (0, 0)),
        ],
        out_shape=[
            jax.ShapeDtypeStruct((NP, H), jnp.float32),
            jax.ShapeDtypeStruct((8, H), jnp.float32),
        ],
    )(z, s, g, be, w)


def _bn_chunks_body(z_ref, s_ref, g_ref, be_ref, o0, o1, o2, o3):
    i = pl.program_id(0)
    scale, shift = _bn_coefs(s_ref, g_ref, be_ref)
    h = jnp.maximum(z_ref[...] * scale + shift, 0.0)
    h = _row_mask(i, h)
    for k, o in enumerate((o0, o1, o2, o3)):
        o[...] = h[:, k * 32:(k + 1) * 32]


def _bn_chunks(z, s, g, be):
    return pl.pallas_call(
        _bn_chunks_body,
        grid=(GRID,),
        in_specs=[
            pl.BlockSpec((BN_ROWS, H), lambda i: (i, 0)),
            pl.BlockSpec((8, H), lambda i: (0, 0)),
            pl.BlockSpec((1, H), lambda i: (0, 0)),
            pl.BlockSpec((1, H), lambda i: (0, 0)),
        ],
        out_specs=[pl.BlockSpec((BN_ROWS, 32), lambda i: (i, 0))] * 4,
        out_shape=[jax.ShapeDtypeStruct((NP, 32), jnp.float32)] * 4,
    )(z, s, g, be)


def _k4_body(h0, h1, h2, h3, a0, a1, a2, a3, b0, b1, b2, b3, w_ref,
             z_ref, s_ref):
    i = pl.program_id(0)
    cols = []
    for hc, ac, bc in ((h0, a0, b0), (h1, a1, b1), (h2, a2, b2), (h3, a3, b3)):
        cols.append(hc[...] + ac[...] + bc[...])
    hin = jnp.concatenate(cols, axis=1)
    z = lax.dot_general(hin, w_ref[...], (((1,), (0,)), ((), ())),
                        preferred_element_type=jnp.float32)
    z_ref[...] = z
    _stats_update(i, z, s_ref)


def _k4(hcs, pa, pb, w):
    return pl.pallas_call(
        _k4_body,
        grid=(GRID,),
        in_specs=(
            [pl.BlockSpec((BN_ROWS, 32), lambda i: (i, 0))] * 12
            + [pl.BlockSpec((H, H), lambda i: (0, 0))]
        ),
        out_specs=[
            pl.BlockSpec((BN_ROWS, H), lambda i: (i, 0)),
            pl.BlockSpec((8, H), lambda i: (0, 0)),
        ],
        out_shape=[
            jax.ShapeDtypeStruct((NP, H), jnp.float32),
            jax.ShapeDtypeStruct((8, H), jnp.float32),
        ],
    )(*hcs, *pa, *pb, w)


def _k6_body(z_ref, s_ref, g_ref, be_ref, w_ref, b_ref, o_ref):
    scale, shift = _bn_coefs(s_ref, g_ref, be_ref)
    h = jnp.maximum(z_ref[...] * scale + shift, 0.0)
    logits = lax.dot_general(h, w_ref[...], (((1,), (0,)), ((), ())),
                             preferred_element_type=jnp.float32) + b_ref[...]
    col = lax.broadcasted_iota(jnp.int32, (1, 8), 1)
    neg = jnp.where(col < NC, logits, -jnp.inf)
    mx = jnp.max(neg, axis=1, keepdims=True)
    e = jnp.where(col < NC, jnp.exp(logits - mx), 0.0)
    lse = mx + jnp.log(jnp.sum(e, axis=1, keepdims=True))
    o_ref[...] = logits - lse


def _k6(z, s, g, be, wf8, bf8):
    return pl.pallas_call(
        _k6_body,
        grid=(GRID,),
        in_specs=[
            pl.BlockSpec((BN_ROWS, H), lambda i: (i, 0)),
            pl.BlockSpec((8, H), lambda i: (0, 0)),
            pl.BlockSpec((1, H), lambda i: (0, 0)),
            pl.BlockSpec((1, H), lambda i: (0, 0)),
            pl.BlockSpec((H, 8), lambda i: (0, 0)),
            pl.BlockSpec((1, 8), lambda i: (0, 0)),
        ],
        out_specs=pl.BlockSpec((BN_ROWS, 8), lambda i: (i, 0)),
        out_shape=jax.ShapeDtypeStruct((NP, 8), jnp.float32),
    )(z, s, g, be, wf8, bf8)


def kernel(x, edge_index, edge_weight, W1a, b1a, g1a, be1a, W2a, b2a, g2a,
           be2a, W1b, b1b, g1b, be1b, W2b, b2b, g2b, be2b, Wf, bf):
    f32 = jnp.float32

    src = edge_index[0]
    dst = edge_index[1]
    pad = EP - E
    pidx = jnp.arange(pad, dtype=jnp.int32)
    src_p = jnp.concatenate([src, pidx % jnp.int32(N)])
    dst_p = jnp.concatenate([dst, jnp.int32(TRASH) + pidx % jnp.int32(NPA - NP)])
    src2d = src_p.reshape(EP // EBLK, EBLK)
    dst2d = dst_p.reshape(EP // EBLK, EBLK)

    x8 = jnp.zeros((NP, 8), f32).at[:N, :NC].set(x)
    w1a8 = jnp.zeros((8, H), f32).at[:NC, :].set(W1a)
    wf8 = jnp.zeros((H, 8), f32).at[:, :NC].set(Wf)
    bf8 = jnp.zeros((1, 8), f32).at[0, :NC].set(bf)
    g1a_, be1a_ = g1a.reshape(1, H), be1a.reshape(1, H)
    g2a_, be2a_ = g2a.reshape(1, H), be2a.reshape(1, H)
    g1b_, be1b_ = g1b.reshape(1, H), be1b.reshape(1, H)
    g2b_, be2b_ = g2b.reshape(1, H), be2b.reshape(1, H)
    zr8 = jnp.zeros((ZR, 8), f32)
    zr32 = jnp.zeros((ZR, 32), f32)

    (p,) = _make_agg(8, 1)(src2d, dst2d, zr8, x8)
    p0, p1 = p[:NP], p[NP:]

    z1, s1 = _k1(x8, p0, p1, w1a8)
    z2, s2 = _bn_mm(z1, s1, g1a_, be1a_, W2a)
    hcs = _bn_chunks(z2, s2, g2a_, be2a_)

    aggs = _make_agg(32, 4)(src2d, dst2d, zr32, *hcs)
    pa = [a[:NP] for a in aggs]
    pb = [a[NP:] for a in aggs]

    z3, s3 = _k4(hcs, pa, pb, W1b)
    z4, s4 = _bn_mm(z3, s3, g1b_, be1b_, W2b)
    out = _k6(z4, s4, g2b_, be2b_, wf8, bf8)
    return out[:N, :NC]
